# baseline (device time: 27734 ns/iter reference)
import jax
import jax.numpy as jnp
from jax import lax
from jax.experimental import pallas as pl
from jax.experimental.pallas import tpu as pltpu


def _exchange(xs):
    m, n = xs.shape

    def body(xs_ref, out_ref, send_sem, recv_sem):
        my_x = lax.axis_index("x")
        my_y = lax.axis_index("y")
        peer = (1 - my_x, my_y)

        barrier = pltpu.get_barrier_semaphore()
        pl.semaphore_signal(
            barrier, inc=1, device_id=peer, device_id_type=pl.DeviceIdType.MESH
        )
        pl.semaphore_wait(barrier, 1)

        rdma = pltpu.make_async_remote_copy(
            src_ref=xs_ref,
            dst_ref=out_ref,
            send_sem=send_sem,
            recv_sem=recv_sem,
            device_id=peer,
            device_id_type=pl.DeviceIdType.MESH,
        )
        rdma.start()
        rdma.wait()

    return pl.pallas_call(
        body,
        out_shape=jax.ShapeDtypeStruct((m, n), xs.dtype),
        in_specs=[pl.BlockSpec(memory_space=pltpu.VMEM)],
        out_specs=pl.BlockSpec(memory_space=pltpu.VMEM),
        scratch_shapes=[pltpu.SemaphoreType.DMA, pltpu.SemaphoreType.DMA],
        compiler_params=pltpu.CompilerParams(collective_id=0),
    )(xs)


def kernel(x, dest):
    m = x.shape[0]
    my_x = lax.axis_index("x")
    c0 = jnp.sum((dest == 0).astype(jnp.int32))
    perm = jnp.argsort(dest, stable=True)
    xs = x[perm].astype(jnp.bfloat16)

    peer_xs = _exchange(xs)

    rolled = jnp.roll(peer_xs, c0, axis=0)
    r = jnp.arange(m)
    keep = jnp.where(my_x == 0, r < c0, r >= c0)
    out = jnp.where(keep[:, None], xs, rolled)
    return out.astype(jnp.float32)
